# baseline (speedup 1.0000x reference)
import functools

import jax
import jax.numpy as jnp
from jax import lax
from jax.experimental import pallas as pl
from jax.experimental.pallas import tpu as pltpu
from jax.experimental.pallas import tpu_sc as plsc

NC, NS, L = 2, 16, 16
NW = NC * NS
K = 32
D = 128
CHUNK = 4
NBUF = 2
A_ROWS = 184
B_ROWS = 456
E_PAD = NS * (A_ROWS + B_ROWS)
PER_MAX = B_ROWS


def _mm_scale_body(x_ref, w1_ref, w3_ref, o_ref):
    w3 = w3_ref[...]
    tv = w3[0:1, :]
    dot = jnp.sum(w3 * tv, axis=1)
    norms = jnp.sqrt(jnp.sum(w3 * w3, axis=1))
    nv = jnp.sqrt(jnp.sum(tv * tv))
    inter = jnp.mean(dot / (nv * norms))
    o_ref[...] = jnp.dot(x_ref[...], w1_ref[...],
                         preferred_element_type=jnp.float32) * inter


def _relu_mm_body(x_ref, w_ref, o_ref):
    o_ref[...] = jnp.dot(jnp.maximum(x_ref[...], 0.0), w_ref[...],
                         preferred_element_type=jnp.float32)


def _tc_mm_scale(x, w1, w3):
    n = x.shape[0]
    blk = 1000
    grid = n // blk
    return pl.pallas_call(
        _mm_scale_body,
        grid=(grid,),
        in_specs=[
            pl.BlockSpec((blk, D), lambda i: (i, 0)),
            pl.BlockSpec((D, D), lambda i: (0, 0)),
            pl.BlockSpec(w3.shape, lambda i: (0, 0)),
        ],
        out_specs=pl.BlockSpec((blk, D), lambda i: (i, 0)),
        out_shape=jax.ShapeDtypeStruct((n, D), jnp.float32),
    )(x, w1, w3)


def _tc_relu_mm(x, w):
    n = x.shape[0]
    blk = 1000
    grid = n // blk
    return pl.pallas_call(
        _relu_mm_body,
        grid=(grid,),
        in_specs=[
            pl.BlockSpec((blk, D), lambda i: (i, 0)),
            pl.BlockSpec((D, D), lambda i: (0, 0)),
        ],
        out_specs=pl.BlockSpec((blk, D), lambda i: (i, 0)),
        out_shape=jax.ShapeDtypeStruct((n, D), jnp.float32),
    )(x, w)


def _sc_body(table_hbm, idx_hbm, out_hbm,
             idx_all, rows, out_v, x0_v, sems):
    c = lax.axis_index("c")
    s = lax.axis_index("s")
    slow = c == 1
    cnt = jnp.where(slow, A_ROWS, B_ROWS)
    base_e = jnp.where(slow, s * A_ROWS, NS * A_ROWS + s * B_ROWS)
    pltpu.sync_copy(idx_hbm.at[pl.ds(base_e * K, PER_MAX * K)], idx_all)
    pltpu.sync_copy(table_hbm.at[pl.ds(0, 1), :], x0_v)

    def issue(g, b):
        src = table_hbm.at[idx_all.at[pl.ds(g * CHUNK * K, CHUNK * K)]]
        pltpu.async_copy(src, rows[b], sems[b])

    for b in range(NBUF):
        issue(b, b)

    def compute(g, b):
        pltpu.make_async_copy(
            table_hbm.at[idx_all.at[pl.ds(0, CHUNK * K)]], rows[b],
            sems[b]).wait()
        base_i = g * (CHUNK * K)
        for j in range(CHUNK):
            zv = jnp.zeros((L,), jnp.int32)
            for h in range(K // L):
                v = idx_all[pl.ds(base_i + j * K + h * L, L)]
                zv = zv + jnp.where(v == 0, 1, 0).astype(jnp.int32)
            lane = lax.iota(jnp.int32, L)
            for sft in (8, 4, 2, 1):
                zv = zv + zv.at[lane ^ sft].get(mode="promise_in_bounds")
            m = K - zv
            pos = m > 0
            scale = 1.0 / jnp.where(pos, m.astype(jnp.float32),
                                    jnp.float32(K))
            zeff = jnp.where(pos, zv.astype(jnp.float32), 0.0)
            rv = rows[b]
            for d in range(D // L):
                sl = pl.ds(d * L, L)
                a0 = rv[j * K + 0, sl]
                a1 = rv[j * K + 1, sl]
                a2 = rv[j * K + 2, sl]
                a3 = rv[j * K + 3, sl]
                for k in range(4, K, 4):
                    a0 = a0 + rv[j * K + k + 0, sl]
                    a1 = a1 + rv[j * K + k + 1, sl]
                    a2 = a2 + rv[j * K + k + 2, sl]
                    a3 = a3 + rv[j * K + k + 3, sl]
                acc = (a0 + a1) + (a2 + a3)
                out_v[g * CHUNK + j, sl] = \
                    (acc - zeff * x0_v[0, sl]) * scale

    n_chunks = cnt // CHUNK

    def group_body(p, carry):
        for b in range(NBUF):
            g = NBUF * p + b
            compute(g, b)

            @pl.when(g + NBUF < n_chunks)
            def _():
                issue(g + NBUF, b)
        return carry

    lax.fori_loop(0, cnt // (CHUNK * NBUF), group_body, 0)

    @pl.when(slow)
    def _():
        pltpu.sync_copy(out_v.at[pl.ds(0, A_ROWS), :],
                        out_hbm.at[pl.ds(base_e, A_ROWS), :])

    @pl.when(jnp.logical_not(slow))
    def _():
        pltpu.sync_copy(out_v.at[pl.ds(0, B_ROWS), :],
                        out_hbm.at[pl.ds(base_e, B_ROWS), :])


def _sc_body_wrap(table_hbm, idx_hbm, out_hbm, idx_all, r0, r1,
                  out_v, x0_v, s0, s1):
    _sc_body(table_hbm, idx_hbm, out_hbm, idx_all, (r0, r1),
             out_v, x0_v, (s0, s1))


def _sc_gather_reduce(table, idx_flat):
    mesh = plsc.VectorSubcoreMesh(core_axis_name="c", subcore_axis_name="s",
                                  num_cores=NC, num_subcores=NS)
    f = pl.kernel(
        _sc_body_wrap,
        out_type=jax.ShapeDtypeStruct((E_PAD, D), jnp.float32),
        mesh=mesh,
        scratch_types=[
            pltpu.VMEM((PER_MAX * K,), jnp.int32),
            pltpu.VMEM((CHUNK * K, D), jnp.float32),
            pltpu.VMEM((CHUNK * K, D), jnp.float32),
            pltpu.VMEM((PER_MAX, D), jnp.float32),
            pltpu.VMEM((1, D), jnp.float32),
            pltpu.SemaphoreType.DMA,
            pltpu.SemaphoreType.DMA,
        ],
    )
    return f(table, idx_flat)


def kernel(x, seq, useq, TextVector, W1, W2, W3):
    n = x.shape[0]
    e = seq.shape[0]
    seq_i = jnp.pad(seq.astype(jnp.int32), ((0, E_PAD - e), (0, 0))).reshape(-1)
    useq_i = jnp.pad(useq.astype(jnp.int32), ((0, E_PAD - n), (0, 0))).reshape(-1)

    x1 = _tc_mm_scale(x, W1, W3)
    edge = _sc_gather_reduce(x1, seq_i)[:e]
    e1 = _tc_relu_mm(edge, W2)
    node = _sc_gather_reduce(e1, useq_i)[:n]
    return node


# speedup vs baseline: 1.0593x; 1.0593x over previous
import functools

import jax
import jax.numpy as jnp
from jax import lax
from jax.experimental import pallas as pl
from jax.experimental.pallas import tpu as pltpu
from jax.experimental.pallas import tpu_sc as plsc

NC, NS, L = 2, 16, 16
NW = NC * NS
K = 32
D = 128
CHUNK = 4
NBUF = 2
A_ROWS = 320
B_ROWS = 320
E_PAD = NS * (A_ROWS + B_ROWS)
PER_MAX = max(A_ROWS, B_ROWS)
N_ROWS = 10000


def _mm_scale_body(x_ref, w1_ref, w3_ref, o_ref):
    w3 = w3_ref[...]
    tv = w3[0:1, :]
    dot = jnp.sum(w3 * tv, axis=1)
    norms = jnp.sqrt(jnp.sum(w3 * w3, axis=1))
    nv = jnp.sqrt(jnp.sum(tv * tv))
    inter = jnp.mean(dot / (nv * norms))
    o_ref[...] = jnp.dot(x_ref[...], w1_ref[...],
                         preferred_element_type=jnp.float32) * inter


def _relu_mm_body(x_ref, w_ref, o_ref):
    o_ref[...] = jnp.dot(jnp.maximum(x_ref[...], 0.0), w_ref[...],
                         preferred_element_type=jnp.float32)


def _tc_mm_scale(x, w1, w3):
    n = x.shape[0]
    blk = 1000
    grid = n // blk
    return pl.pallas_call(
        _mm_scale_body,
        grid=(grid,),
        in_specs=[
            pl.BlockSpec((blk, D), lambda i: (i, 0)),
            pl.BlockSpec((D, D), lambda i: (0, 0)),
            pl.BlockSpec(w3.shape, lambda i: (0, 0)),
        ],
        out_specs=pl.BlockSpec((blk, D), lambda i: (i, 0)),
        out_shape=jax.ShapeDtypeStruct((n, D), jnp.float32),
    )(x, w1, w3)


def _tc_relu_mm(x, w):
    n = x.shape[0]
    blk = 1000
    grid = n // blk
    return pl.pallas_call(
        _relu_mm_body,
        grid=(grid,),
        in_specs=[
            pl.BlockSpec((blk, D), lambda i: (i, 0)),
            pl.BlockSpec((D, D), lambda i: (0, 0)),
        ],
        out_specs=pl.BlockSpec((blk, D), lambda i: (i, 0)),
        out_shape=jax.ShapeDtypeStruct((n, D), jnp.float32),
    )(x, w)


def _sc_body(table_hbm, idx_hbm, out_hbm,
             idx_all, rows, out_v, x0_v, sems):
    c = lax.axis_index("c")
    s = lax.axis_index("s")
    slow = c == 1
    cnt = jnp.where(slow, A_ROWS, B_ROWS)
    base_e = jnp.where(slow, s * A_ROWS, NS * A_ROWS + s * B_ROWS)
    pltpu.sync_copy(idx_hbm.at[pl.ds(base_e * K, PER_MAX * K)], idx_all)
    pltpu.sync_copy(table_hbm.at[pl.ds(0, 1), :], x0_v)

    off = (c * N_ROWS).astype(jnp.int32)

    def off_body(i, carry):
        idx_all[pl.ds(i * L, L)] = idx_all[pl.ds(i * L, L)] + off
        return carry

    lax.fori_loop(0, PER_MAX * K // L, off_body, 0)

    def issue(g, b):
        src = table_hbm.at[idx_all.at[pl.ds(g * CHUNK * K, CHUNK * K)]]
        pltpu.async_copy(src, rows[b], sems[b])

    for b in range(NBUF):
        issue(b, b)

    def compute(g, b):
        pltpu.make_async_copy(
            table_hbm.at[idx_all.at[pl.ds(0, CHUNK * K)]], rows[b],
            sems[b]).wait()
        base_i = g * (CHUNK * K)
        for j in range(CHUNK):
            zv = jnp.zeros((L,), jnp.int32)
            for h in range(K // L):
                v = idx_all[pl.ds(base_i + j * K + h * L, L)]
                zv = zv + jnp.where(v == off, 1, 0).astype(jnp.int32)
            lane = lax.iota(jnp.int32, L)
            for sft in (8, 4, 2, 1):
                zv = zv + zv.at[lane ^ sft].get(mode="promise_in_bounds")
            m = K - zv
            pos = m > 0
            scale = 1.0 / jnp.where(pos, m.astype(jnp.float32),
                                    jnp.float32(K))
            zeff = jnp.where(pos, zv.astype(jnp.float32), 0.0)
            rv = rows[b]
            for d in range(D // L):
                sl = pl.ds(d * L, L)
                a0 = rv[j * K + 0, sl]
                a1 = rv[j * K + 1, sl]
                a2 = rv[j * K + 2, sl]
                a3 = rv[j * K + 3, sl]
                for k in range(4, K, 4):
                    a0 = a0 + rv[j * K + k + 0, sl]
                    a1 = a1 + rv[j * K + k + 1, sl]
                    a2 = a2 + rv[j * K + k + 2, sl]
                    a3 = a3 + rv[j * K + k + 3, sl]
                acc = (a0 + a1) + (a2 + a3)
                out_v[g * CHUNK + j, sl] = \
                    (acc - zeff * x0_v[0, sl]) * scale

    n_chunks = cnt // CHUNK

    def group_body(p, carry):
        for b in range(NBUF):
            g = NBUF * p + b
            compute(g, b)

            @pl.when(g + NBUF < n_chunks)
            def _():
                issue(g + NBUF, b)
        return carry

    lax.fori_loop(0, cnt // (CHUNK * NBUF), group_body, 0)

    @pl.when(slow)
    def _():
        pltpu.sync_copy(out_v.at[pl.ds(0, A_ROWS), :],
                        out_hbm.at[pl.ds(base_e, A_ROWS), :])

    @pl.when(jnp.logical_not(slow))
    def _():
        pltpu.sync_copy(out_v.at[pl.ds(0, B_ROWS), :],
                        out_hbm.at[pl.ds(base_e, B_ROWS), :])


def _sc_body_wrap(table_hbm, idx_hbm, out_hbm, idx_all, r0, r1,
                  out_v, x0_v, s0, s1):
    _sc_body(table_hbm, idx_hbm, out_hbm, idx_all, (r0, r1),
             out_v, x0_v, (s0, s1))


def _sc_gather_reduce(table, idx_flat):
    mesh = plsc.VectorSubcoreMesh(core_axis_name="c", subcore_axis_name="s",
                                  num_cores=NC, num_subcores=NS)
    f = pl.kernel(
        _sc_body_wrap,
        out_type=jax.ShapeDtypeStruct((E_PAD, D), jnp.float32),
        mesh=mesh,
        scratch_types=[
            pltpu.VMEM((PER_MAX * K,), jnp.int32),
            pltpu.VMEM((CHUNK * K, D), jnp.float32),
            pltpu.VMEM((CHUNK * K, D), jnp.float32),
            pltpu.VMEM((PER_MAX, D), jnp.float32),
            pltpu.VMEM((1, D), jnp.float32),
            pltpu.SemaphoreType.DMA,
            pltpu.SemaphoreType.DMA,
        ],
    )
    return f(table, idx_flat)


def kernel(x, seq, useq, TextVector, W1, W2, W3):
    n = x.shape[0]
    e = seq.shape[0]
    seq_i = jnp.pad(seq.astype(jnp.int32), ((0, E_PAD - e), (0, 0))).reshape(-1)
    useq_i = jnp.pad(useq.astype(jnp.int32), ((0, E_PAD - n), (0, 0))).reshape(-1)

    x1 = _tc_mm_scale(x, W1, W3)
    edge = _sc_gather_reduce(jnp.concatenate([x1, x1], axis=0), seq_i)[:e]
    e1 = _tc_relu_mm(edge, W2)
    node = _sc_gather_reduce(jnp.concatenate([e1, e1], axis=0), useq_i)[:n]
    return node


# speedup vs baseline: 4.0547x; 3.8279x over previous
import functools

import jax
import jax.numpy as jnp
from jax import lax
from jax.experimental import pallas as pl
from jax.experimental.pallas import tpu as pltpu
from jax.experimental.pallas import tpu_sc as plsc

NC, NS, L = 2, 16, 16
NW = NC * NS
K = 32
D = 128
CHUNK = 2
NBUF = 2
PER_W = 320
E_PAD = NW * PER_W
N_CHUNKS = PER_W // CHUNK
N_ROWS = 10000
RPS = 624


def _mm_scale_body(x_ref, w1_ref, w3_ref, o_ref):
    w3 = w3_ref[...]
    tv = w3[0:1, :]
    dot = jnp.sum(w3 * tv, axis=1)
    norms = jnp.sqrt(jnp.sum(w3 * w3, axis=1))
    nv = jnp.sqrt(jnp.sum(tv * tv))
    inter = jnp.mean(dot / (nv * norms))
    o_ref[...] = jnp.dot(x_ref[...], w1_ref[...],
                         preferred_element_type=jnp.float32) * inter


def _relu_mm_body(x_ref, w_ref, o_ref):
    o_ref[...] = jnp.dot(jnp.maximum(x_ref[...], 0.0), w_ref[...],
                         preferred_element_type=jnp.float32)


def _tc_mm_scale(x, w1, w3):
    n = x.shape[0]
    blk = 1000
    grid = n // blk
    return pl.pallas_call(
        _mm_scale_body,
        grid=(grid,),
        in_specs=[
            pl.BlockSpec((blk, D), lambda i: (i, 0)),
            pl.BlockSpec((D, D), lambda i: (0, 0)),
            pl.BlockSpec(w3.shape, lambda i: (0, 0)),
        ],
        out_specs=pl.BlockSpec((blk, D), lambda i: (i, 0)),
        out_shape=jax.ShapeDtypeStruct((n, D), jnp.float32),
    )(x, w1, w3)


def _tc_relu_mm(x, w):
    n = x.shape[0]
    blk = 1000
    grid = n // blk
    return pl.pallas_call(
        _relu_mm_body,
        grid=(grid,),
        in_specs=[
            pl.BlockSpec((blk, D), lambda i: (i, 0)),
            pl.BlockSpec((D, D), lambda i: (0, 0)),
        ],
        out_specs=pl.BlockSpec((blk, D), lambda i: (i, 0)),
        out_shape=jax.ShapeDtypeStruct((n, D), jnp.float32),
    )(x, w)


def _sc_body(table_hbm, idx_hbm, out_hbm,
             idx_all, rows, obufs, x0_v, shared, sems, osems):
    c = lax.axis_index("c")
    s = lax.axis_index("s")
    wid = s * NC + c
    base_e = wid * PER_W
    pltpu.sync_copy(table_hbm.at[pl.ds(s * RPS, RPS), :],
                    shared.at[pl.ds(s * RPS, RPS), :])

    @pl.when(s == 0)
    def _():
        pltpu.sync_copy(table_hbm.at[pl.ds(NS * RPS, N_ROWS - NS * RPS), :],
                        shared.at[pl.ds(NS * RPS, N_ROWS - NS * RPS), :])

    pltpu.sync_copy(idx_hbm.at[pl.ds(base_e * K, PER_W * K)], idx_all)
    pltpu.sync_copy(table_hbm.at[pl.ds(0, 1), :], x0_v)
    plsc.subcore_barrier()

    def issue(g, b):
        src = shared.at[idx_all.at[pl.ds(g * CHUNK * K, CHUNK * K)]]
        pltpu.async_copy(src, rows[b], sems[b])

    for b in range(NBUF):
        issue(b, b)

    def compute(g, b):
        pltpu.make_async_copy(
            shared.at[idx_all.at[pl.ds(0, CHUNK * K)]], rows[b],
            sems[b]).wait()

        @pl.when(g >= NBUF)
        def _():
            pltpu.make_async_copy(
                obufs[b], out_hbm.at[pl.ds(0, CHUNK), :], osems[b]).wait()

        base_i = g * (CHUNK * K)
        for j in range(CHUNK):
            zv = jnp.zeros((L,), jnp.int32)
            for h in range(K // L):
                v = idx_all[pl.ds(base_i + j * K + h * L, L)]
                zv = zv + jnp.where(v == 0, 1, 0).astype(jnp.int32)
            lane = lax.iota(jnp.int32, L)
            for sft in (8, 4, 2, 1):
                zv = zv + zv.at[lane ^ sft].get(mode="promise_in_bounds")
            m = K - zv
            pos = m > 0
            scale = 1.0 / jnp.where(pos, m.astype(jnp.float32),
                                    jnp.float32(K))
            zeff = jnp.where(pos, zv.astype(jnp.float32), 0.0)
            rv = rows[b]
            for d in range(D // L):
                sl = pl.ds(d * L, L)
                a0 = rv[j * K + 0, sl]
                a1 = rv[j * K + 1, sl]
                a2 = rv[j * K + 2, sl]
                a3 = rv[j * K + 3, sl]
                for k in range(4, K, 4):
                    a0 = a0 + rv[j * K + k + 0, sl]
                    a1 = a1 + rv[j * K + k + 1, sl]
                    a2 = a2 + rv[j * K + k + 2, sl]
                    a3 = a3 + rv[j * K + k + 3, sl]
                acc = (a0 + a1) + (a2 + a3)
                obufs[b][j, sl] = (acc - zeff * x0_v[0, sl]) * scale
        pltpu.async_copy(obufs[b],
                         out_hbm.at[pl.ds(base_e + g * CHUNK, CHUNK), :],
                         osems[b])

    def group_body(p, carry):
        for b in range(NBUF):
            g = NBUF * p + b
            compute(g, b)

            @pl.when(g + NBUF < N_CHUNKS)
            def _():
                issue(g + NBUF, b)
        return carry

    lax.fori_loop(0, N_CHUNKS // NBUF, group_body, 0)
    for b in range(NBUF):
        pltpu.make_async_copy(
            obufs[b], out_hbm.at[pl.ds(0, CHUNK), :], osems[b]).wait()


def _sc_body_wrap(table_hbm, idx_hbm, out_hbm, idx_all, r0, r1, o0, o1,
                  x0_v, shared, s0, s1, os0, os1):
    _sc_body(table_hbm, idx_hbm, out_hbm, idx_all, (r0, r1), (o0, o1),
             x0_v, shared, (s0, s1), (os0, os1))


def _sc_gather_reduce(table, idx_flat):
    mesh = plsc.VectorSubcoreMesh(core_axis_name="c", subcore_axis_name="s",
                                  num_cores=NC, num_subcores=NS)
    f = pl.kernel(
        _sc_body_wrap,
        out_type=jax.ShapeDtypeStruct((E_PAD, D), jnp.float32),
        mesh=mesh,
        scratch_types=[
            pltpu.VMEM((PER_W * K,), jnp.int32),
            pltpu.VMEM((CHUNK * K, D), jnp.float32),
            pltpu.VMEM((CHUNK * K, D), jnp.float32),
            pltpu.VMEM((CHUNK, D), jnp.float32),
            pltpu.VMEM((CHUNK, D), jnp.float32),
            pltpu.VMEM((1, D), jnp.float32),
            pltpu.VMEM_SHARED((N_ROWS, D), jnp.float32),
            pltpu.SemaphoreType.DMA,
            pltpu.SemaphoreType.DMA,
            pltpu.SemaphoreType.DMA,
            pltpu.SemaphoreType.DMA,
        ],
    )
    return f(table, idx_flat)


def kernel(x, seq, useq, TextVector, W1, W2, W3):
    n = x.shape[0]
    e = seq.shape[0]
    seq_i = jnp.pad(seq.astype(jnp.int32), ((0, E_PAD - e), (0, 0))).reshape(-1)
    useq_i = jnp.pad(useq.astype(jnp.int32), ((0, E_PAD - n), (0, 0))).reshape(-1)

    x1 = _tc_mm_scale(x, W1, W3)
    edge = _sc_gather_reduce(x1, seq_i)[:e]
    e1 = _tc_relu_mm(edge, W2)
    node = _sc_gather_reduce(e1, useq_i)[:n]
    return node
